# Initial kernel scaffold; baseline (speedup 1.0000x reference)
import jax
import jax.numpy as jnp
from jax.experimental import pallas as pl


def kernel(node_data, node_num, lap_eigvec, edge_index, edge_data, edge_num, atom_emb, edge_emb, lap_W, order_emb):
    raise NotImplementedError("write your pallas kernel here")


# speedup vs baseline: 46.6058x; 46.6058x over previous
import jax
import jax.numpy as jnp
from jax import lax
from jax.experimental import pallas as pl
from jax.experimental.pallas import tpu as pltpu
from jax.experimental.pallas import tpu_sc as plsc

B = 8
NN = 512
EN = 1024
K = 16
D = 768
V = 8192
MAXLEN = 2048
ACT = 1024
NR = B * NN
C = 32
NCH = 4
NV = D // 16


def _tc_body(lap_ref, w0t_ref, w1t_ref, ord_ref, p_ref, q3_ref):
    a = lap_ref[...]
    p = jnp.dot(a, w0t_ref[...], preferred_element_type=jnp.float32)
    q = jnp.dot(a, w1t_ref[...], preferred_element_type=jnp.float32)
    p_ref[...] = p
    q3_ref[0] = q + ord_ref[0:1, :]
    q3_ref[1] = q + ord_ref[1:2, :]


def _sc_body(nfid_h, efid_h, egu_h, egq_h, atom_h, edge_h, p_h, q3_h, out_h,
             idxa, idxe, idxu, idxq, buf0, buf1, buf2, s0, s1, s2):
    cid = lax.axis_index("c")
    sid = lax.axis_index("s")
    wid = sid * 2 + cid
    b = wid // 4
    qq = wid % 4
    nbase = wid * (NCH * C)
    irow = wid * NCH
    nout = b * MAXLEN + qq * (NCH * C)
    eout = nout + NN

    pltpu.sync_copy(nfid_h.at[pl.ds(irow, NCH)], idxa)
    pltpu.sync_copy(efid_h.at[pl.ds(irow, NCH)], idxe)
    pltpu.sync_copy(egu_h.at[pl.ds(irow, NCH)], idxu)
    pltpu.sync_copy(egq_h.at[pl.ds(irow, NCH)], idxq)

    def _accum(_t, carry):
        for vv in range(NV):
            sl = pl.ds(vv * 16, 16)
            buf0[_t, sl] = buf0[_t, sl] + buf1[_t, sl] + buf2[_t, sl]
        return carry

    for c in range(NCH):
        ca = pltpu.async_copy(atom_h.at[idxa.at[c]], buf0, s0)
        cp = pltpu.async_copy(p_h.at[pl.ds(nbase + c * C, C)], buf1, s1)
        cq = pltpu.async_copy(q3_h.at[pl.ds(NR + nbase + c * C, C)], buf2, s2)
        ca.wait()
        cp.wait()
        cq.wait()
        lax.fori_loop(0, C, _accum, 0)
        pltpu.sync_copy(buf0, out_h.at[pl.ds(nout + c * C, C)])

    for c in range(NCH):
        ca = pltpu.async_copy(edge_h.at[idxe.at[c]], buf0, s0)
        cp = pltpu.async_copy(p_h.at[idxu.at[c]], buf1, s1)
        cq = pltpu.async_copy(q3_h.at[idxq.at[c]], buf2, s2)
        ca.wait()
        cp.wait()
        cq.wait()
        lax.fori_loop(0, C, _accum, 0)
        pltpu.sync_copy(buf0, out_h.at[pl.ds(eout + c * C, C)])

    def _zero(_t, carry):
        for vv in range(NV):
            buf0[_t, pl.ds(vv * 16, 16)] = jnp.zeros((16,), jnp.float32)
        return carry

    lax.fori_loop(0, C, _zero, 0)
    zbase = b * MAXLEN + ACT + qq * (2 * NCH * C)
    for c in range(2 * NCH):
        pltpu.sync_copy(buf0, out_h.at[pl.ds(zbase + c * C, C)])


def kernel(node_data, node_num, lap_eigvec, edge_index, edge_data, edge_num,
           atom_emb, edge_emb, lap_W, order_emb):
    nfid = node_data.reshape(NR // C, C).astype(jnp.int32)
    efid = edge_data.reshape(B, EN)[:, :NN].reshape(NR // C, C).astype(jnp.int32)

    ei = edge_index.astype(jnp.int32)
    off = (jnp.arange(B, dtype=jnp.int32) * NN)[:, None]
    eu = ei[0].reshape(B, EN)[:, :NN] + off
    ev = ei[1].reshape(B, EN)[:, :NN] + off
    egq = (ev + jnp.where(eu == ev, NR, 0).astype(jnp.int32)).reshape(NR // C, C)
    egu = eu.reshape(NR // C, C)

    lapf = lap_eigvec.astype(jnp.float32)
    w0t = lap_W[:, :K].T
    w1t = lap_W[:, K:].T
    ordm = order_emb.astype(jnp.float32)

    P, Q3 = pl.pallas_call(
        _tc_body,
        grid=(16,),
        in_specs=[
            pl.BlockSpec((256, K), lambda i: (i, 0)),
            pl.BlockSpec((K, D), lambda i: (0, 0)),
            pl.BlockSpec((K, D), lambda i: (0, 0)),
            pl.BlockSpec((2, D), lambda i: (0, 0)),
        ],
        out_specs=[
            pl.BlockSpec((256, D), lambda i: (i, 0)),
            pl.BlockSpec((2, 256, D), lambda i: (0, i, 0)),
        ],
        out_shape=[
            jax.ShapeDtypeStruct((NR, D), jnp.float32),
            jax.ShapeDtypeStruct((2, NR, D), jnp.float32),
        ],
    )(lapf, w0t, w1t, ordm)
    Q3 = Q3.reshape(2 * NR, D)

    mesh = plsc.VectorSubcoreMesh(core_axis_name="c", subcore_axis_name="s")
    outflat = pl.kernel(
        _sc_body,
        out_type=jax.ShapeDtypeStruct((B * MAXLEN, D), jnp.float32),
        mesh=mesh,
        scratch_types=[
            pltpu.VMEM((NCH, C), jnp.int32),
            pltpu.VMEM((NCH, C), jnp.int32),
            pltpu.VMEM((NCH, C), jnp.int32),
            pltpu.VMEM((NCH, C), jnp.int32),
            pltpu.VMEM((C, D), jnp.float32),
            pltpu.VMEM((C, D), jnp.float32),
            pltpu.VMEM((C, D), jnp.float32),
            pltpu.SemaphoreType.DMA,
            pltpu.SemaphoreType.DMA,
            pltpu.SemaphoreType.DMA,
        ],
    )(nfid, efid, egu, egq, atom_emb, edge_emb, P, Q3)

    return outflat.reshape(B, MAXLEN, D)
